# baseline (speedup 1.0000x reference)
import functools

import jax
import jax.numpy as jnp
from jax import lax
from jax.experimental import pallas as pl
from jax.experimental.pallas import tpu as pltpu
from jax.experimental.pallas import tpu_sc as plsc

B = 16384
F = 26
V = 100000
NNUM = 13
H = 128
OUT = 2

NC = 2
NS = 16
NW = NC * NS
FH = F // 2
N_WORDS = B * F
W_PER_T = N_WORDS // NW
CH = 128
NCH = W_PER_T // CH


def _gather_body(table_hbm, idx_hbm, out_hbm, idx_v, rows_v, sem, sem_out, sem_idx):
    wid = lax.axis_index("s") * NC + lax.axis_index("c")

    def fire_idx(j, _):
        pltpu.async_copy(idx_hbm.at[wid].at[j], idx_v.at[j], sem_idx)
        return 0

    def fire(j, _):
        pltpu.make_async_copy(idx_hbm.at[wid].at[j], idx_v.at[j], sem_idx).wait()
        pltpu.async_copy(table_hbm.at[idx_v.at[j]], rows_v.at[j], sem)
        return 0

    def drain_and_store(j, _):
        pltpu.make_async_copy(table_hbm.at[idx_v.at[j]], rows_v.at[j], sem).wait()
        pltpu.async_copy(rows_v.at[j], out_hbm.at[wid].at[j], sem_out)
        return 0

    def drain_out(j, _):
        pltpu.make_async_copy(rows_v.at[j], out_hbm.at[wid].at[j], sem_out).wait()
        return 0

    lax.fori_loop(0, NCH, fire_idx, 0)
    lax.fori_loop(0, NCH, fire, 0)
    lax.fori_loop(0, NCH, drain_and_store, 0)
    lax.fori_loop(0, NCH, drain_out, 0)


def _sc_gather(table_1d, idx3):
    mesh = plsc.VectorSubcoreMesh(core_axis_name="c", subcore_axis_name="s")
    run = pl.kernel(
        _gather_body,
        out_type=jax.ShapeDtypeStruct((NW, NCH, CH), jnp.float32),
        mesh=mesh,
        scratch_types=[
            pltpu.VMEM((NCH, CH), jnp.int32),
            pltpu.VMEM((NCH, CH), jnp.float32),
            pltpu.SemaphoreType.DMA,
            pltpu.SemaphoreType.DMA,
            pltpu.SemaphoreType.DMA,
        ],
        compiler_params=pltpu.CompilerParams(use_tc_tiling_on_sc=False),
    )
    return run(table_1d, idx3)


BB = 4096


def _mlp_body(xn_ref, xa_ref, xb_ref, w1n_ref, w1a_ref, w1b_ref, b1_ref,
              w2_ref, b2_ref, o_ref):
    dn = (((0,), (0,)), ((), ()))
    h = lax.dot_general(xn_ref[...], w1n_ref[...], dn,
                        preferred_element_type=jnp.float32)
    h = h + lax.dot_general(xa_ref[...], w1a_ref[...], dn,
                            preferred_element_type=jnp.float32)
    h = h + lax.dot_general(xb_ref[...], w1b_ref[...], dn,
                            preferred_element_type=jnp.float32)
    h = jnp.maximum(h + b1_ref[...], 0.0)
    yt = lax.dot_general(w2_ref[...], h, (((0,), (1,)), ((), ())),
                         preferred_element_type=jnp.float32) + b2_ref[...]
    row = lax.broadcasted_iota(jnp.int32, yt.shape, 0)
    o_ref[...] = jnp.where(row == 0, yt, jnp.maximum(yt, 0.0) + 0.025)


def _tc_mlp(x_num, xe_a, xe_b, W1, b1, W2, b2):
    w1n = W1[:NNUM]
    w1a = W1[NNUM:NNUM + F]
    w1b = W1[NNUM + F:]
    grid = (B // BB,)
    return pl.pallas_call(
        _mlp_body,
        grid=grid,
        in_specs=[
            pl.BlockSpec((NNUM, BB), lambda i: (0, i)),
            pl.BlockSpec((F, BB), lambda i: (0, i)),
            pl.BlockSpec((F, BB), lambda i: (0, i)),
            pl.BlockSpec((NNUM, H), lambda i: (0, 0)),
            pl.BlockSpec((F, H), lambda i: (0, 0)),
            pl.BlockSpec((F, H), lambda i: (0, 0)),
            pl.BlockSpec((1, H), lambda i: (0, 0)),
            pl.BlockSpec((H, OUT), lambda i: (0, 0)),
            pl.BlockSpec((OUT, 1), lambda i: (0, 0)),
        ],
        out_specs=pl.BlockSpec((OUT, BB), lambda i: (0, i)),
        out_shape=jax.ShapeDtypeStruct((OUT, B), jnp.float32),
    )(x_num.T, xe_a, xe_b, w1n, w1a, w1b, b1.reshape(1, H), W2,
      b2.reshape(OUT, 1)).T


def kernel(x_num, x_cat, tables, W1, b1, W2, b2):
    cols = jnp.arange(2 * F, dtype=jnp.int32)
    idxT = jnp.take(x_cat.T, cols // 2, axis=0) + (cols % (2 * FH))[:, None] * V
    idx_a = idxT[:2 * FH].reshape(NW, NCH, CH)
    idx_b = idxT[2 * FH:].reshape(NW, NCH, CH)
    t_a = tables[:FH].transpose(0, 2, 1).reshape(FH * 2 * V)
    tables_b, _ = lax.optimization_barrier((tables, idx_a))
    t_b = tables_b[FH:].transpose(0, 2, 1).reshape(FH * 2 * V)
    emb_a = _sc_gather(t_a, idx_a)
    emb_b = _sc_gather(t_b, idx_b)
    xe_a = emb_a.reshape(2 * FH, B)
    xe_b = emb_b.reshape(2 * FH, B)
    return _tc_mlp(x_num, xe_a, xe_b, W1, b1, W2, b2)


# speedup vs baseline: 1.0199x; 1.0199x over previous
import functools

import jax
import jax.numpy as jnp
from jax import lax
from jax.experimental import pallas as pl
from jax.experimental.pallas import tpu as pltpu
from jax.experimental.pallas import tpu_sc as plsc

B = 16384
F = 26
V = 100000
NNUM = 13
H = 128
OUT = 2

NC = 2
NS = 16
NW = NC * NS
FH = F // 2
N_WORDS = B * F
W_PER_T = N_WORDS // NW
CH = 128
NCH = W_PER_T // CH


def _gather_body(table_hbm, idx_hbm, out_hbm, idx_v, rows_v, sem, sem_out, sem_idx):
    wid = lax.axis_index("s") * NC + lax.axis_index("c")

    def fire_idx(j, _):
        pltpu.async_copy(idx_hbm.at[wid].at[j], idx_v.at[j], sem_idx)
        return 0

    def fire(j, _):
        pltpu.make_async_copy(idx_hbm.at[wid].at[j], idx_v.at[j], sem_idx).wait()
        pltpu.async_copy(table_hbm.at[idx_v.at[j]], rows_v.at[j], sem)
        return 0

    def drain_and_store(j, _):
        pltpu.make_async_copy(table_hbm.at[idx_v.at[j]], rows_v.at[j], sem).wait()
        pltpu.async_copy(rows_v.at[j], out_hbm.at[wid].at[j], sem_out)
        return 0

    def drain_out(j, _):
        pltpu.make_async_copy(rows_v.at[j], out_hbm.at[wid].at[j], sem_out).wait()
        return 0

    lax.fori_loop(0, NCH, fire_idx, 0)
    lax.fori_loop(0, NCH, fire, 0)
    lax.fori_loop(0, NCH, drain_and_store, 0)
    lax.fori_loop(0, NCH, drain_out, 0)


def _sc_gather(table_1d, idx3):
    mesh = plsc.VectorSubcoreMesh(core_axis_name="c", subcore_axis_name="s")
    run = pl.kernel(
        _gather_body,
        out_type=jax.ShapeDtypeStruct((NW, NCH, CH), jnp.float32),
        mesh=mesh,
        scratch_types=[
            pltpu.VMEM((NCH, CH), jnp.int32),
            pltpu.VMEM((NCH, CH), jnp.float32),
            pltpu.SemaphoreType.DMA,
            pltpu.SemaphoreType.DMA,
            pltpu.SemaphoreType.DMA,
        ],
        compiler_params=pltpu.CompilerParams(use_tc_tiling_on_sc=False),
    )
    return run(table_1d, idx3)


BB = 8192


def _mlp_body(xn_ref, xa_ref, xb_ref, w1n_ref, w1a_ref, w1b_ref, b1_ref,
              w2_ref, b2_ref, o_ref):
    dn = (((0,), (0,)), ((), ()))
    h = lax.dot_general(xn_ref[...], w1n_ref[...], dn,
                        preferred_element_type=jnp.float32)
    h = h + lax.dot_general(xa_ref[...], w1a_ref[...], dn,
                            preferred_element_type=jnp.float32)
    h = h + lax.dot_general(xb_ref[...], w1b_ref[...], dn,
                            preferred_element_type=jnp.float32)
    h = jnp.maximum(h + b1_ref[...], 0.0)
    yt = lax.dot_general(w2_ref[...], h, (((0,), (1,)), ((), ())),
                         preferred_element_type=jnp.float32) + b2_ref[...]
    row = lax.broadcasted_iota(jnp.int32, yt.shape, 0)
    o_ref[...] = jnp.where(row == 0, yt, jnp.maximum(yt, 0.0) + 0.025)


def _tc_mlp(x_num, xe_a, xe_b, W1, b1, W2, b2):
    w1n = W1[:NNUM]
    w1a = W1[NNUM:NNUM + F]
    w1b = W1[NNUM + F:]
    grid = (B // BB,)
    return pl.pallas_call(
        _mlp_body,
        grid=grid,
        in_specs=[
            pl.BlockSpec((NNUM, BB), lambda i: (0, i)),
            pl.BlockSpec((F, BB), lambda i: (0, i)),
            pl.BlockSpec((F, BB), lambda i: (0, i)),
            pl.BlockSpec((NNUM, H), lambda i: (0, 0)),
            pl.BlockSpec((F, H), lambda i: (0, 0)),
            pl.BlockSpec((F, H), lambda i: (0, 0)),
            pl.BlockSpec((1, H), lambda i: (0, 0)),
            pl.BlockSpec((H, OUT), lambda i: (0, 0)),
            pl.BlockSpec((OUT, 1), lambda i: (0, 0)),
        ],
        out_specs=pl.BlockSpec((OUT, BB), lambda i: (0, i)),
        out_shape=jax.ShapeDtypeStruct((OUT, B), jnp.float32),
    )(x_num.T, xe_a, xe_b, w1n, w1a, w1b, b1.reshape(1, H), W2,
      b2.reshape(OUT, 1)).T


def kernel(x_num, x_cat, tables, W1, b1, W2, b2):
    cols = jnp.arange(2 * F, dtype=jnp.int32)
    idxT = jnp.take(x_cat.T, cols // 2, axis=0) + (cols % (2 * FH))[:, None] * V
    idx_a = idxT[:2 * FH].reshape(NW, NCH, CH)
    idx_b = idxT[2 * FH:].reshape(NW, NCH, CH)
    t_a = tables[:FH].transpose(0, 2, 1).reshape(FH * 2 * V)
    t_b = tables[FH:].transpose(0, 2, 1).reshape(FH * 2 * V)
    emb_a = _sc_gather(t_a, idx_a)
    emb_b = _sc_gather(t_b, idx_b)
    xe_a = emb_a.reshape(2 * FH, B)
    xe_b = emb_b.reshape(2 * FH, B)
    return _tc_mlp(x_num, xe_a, xe_b, W1, b1, W2, b2)
